# baseline (speedup 1.0000x reference)
import functools

import jax
import jax.numpy as jnp
from jax import lax
from jax.experimental import pallas as pl
from jax.experimental.pallas import tpu as pltpu
from jax.experimental.pallas import tpu_sc as plsc

N_NODES = 10000
N_EDGES = 320000
H_SIZE = 128

NUM_CORES = 2
NUM_SUBCORES = 16
CHUNK = 128
GR_CHUNKS = 16
GROUPS = 10
CHUNKS_PER_SUBCORE = GR_CHUNKS * GROUPS
EDGES_PER_SUBCORE = CHUNK * CHUNKS_PER_SUBCORE
E_PAD = EDGES_PER_SUBCORE * NUM_SUBCORES
ACC_ROWS = 10240
ZERO_ROWS = ACC_ROWS // NUM_SUBCORES
OUT_ROWS = 624


def _make_segment_sums():
    mesh = plsc.VectorSubcoreMesh(core_axis_name="c", subcore_axis_name="s")

    @functools.partial(
        pl.kernel,
        mesh=mesh,
        out_type=(
            jax.ShapeDtypeStruct((N_NODES, H_SIZE), jnp.float32),
            jax.ShapeDtypeStruct((N_NODES, H_SIZE), jnp.float32),
        ),
        scratch_types=[
            pltpu.VMEM((CHUNK,), jnp.int32),
            pltpu.VMEM((CHUNK,), jnp.int32),
            pltpu.VMEM((CHUNK, H_SIZE), jnp.float32),
            pltpu.VMEM_SHARED((ACC_ROWS, H_SIZE), jnp.float32),
            pltpu.SemaphoreType.DMA,
        ],
    )
    def seg_sum(h_hbm, c_hbm, src_hbm, dst_hbm, zeros_hbm,
                hsum_hbm, csum_hbm, src_v, dst_v, rows_v, acc, sem):
        cid = lax.axis_index("c")
        sid = lax.axis_index("s")

        pltpu.sync_copy(zeros_hbm, acc.at[pl.ds(sid * ZERO_ROWS, ZERO_ROWS)])
        plsc.subcore_barrier()

        def run_edges(table_hbm):
            @pl.loop(0, CHUNKS_PER_SUBCORE)
            def _(i):
                base = sid * EDGES_PER_SUBCORE + i * CHUNK
                pltpu.sync_copy(src_hbm.at[pl.ds(base, CHUNK)], src_v)
                pltpu.sync_copy(dst_hbm.at[pl.ds(base, CHUNK)], dst_v)
                pltpu.async_copy(table_hbm.at[src_v], rows_v, sem).wait()
                pltpu.sync_copy(rows_v, acc.at[dst_v], add=True)

        @pl.when(cid == 0)
        def _():
            run_edges(h_hbm)

        @pl.when(cid == 1)
        def _():
            run_edges(c_hbm)

        plsc.subcore_barrier()

        def writeout(dst_hbm_ref):
            @pl.when(sid < NUM_SUBCORES - 1)
            def _():
                slc = pl.ds(sid * OUT_ROWS, OUT_ROWS)
                pltpu.sync_copy(acc.at[slc], dst_hbm_ref.at[slc])

            @pl.when(sid == NUM_SUBCORES - 1)
            def _():
                slc = pl.ds((NUM_SUBCORES - 1) * OUT_ROWS,
                            N_NODES - (NUM_SUBCORES - 1) * OUT_ROWS)
                pltpu.sync_copy(acc.at[slc], dst_hbm_ref.at[slc])

        @pl.when(cid == 0)
        def _():
            writeout(hsum_hbm)

        @pl.when(cid == 1)
        def _():
            writeout(csum_hbm)

    return seg_sum


_segment_sums = _make_segment_sums()


def _gates_body(hs_ref, cs_ref, wf_ref, bf_ref, wiou_ref, biou_ref,
                hn_ref, cn_ref):
    hs = hs_ref[...]
    f = jax.nn.sigmoid(
        jnp.dot(hs, wf_ref[...], preferred_element_type=jnp.float32)
        + bf_ref[...])
    c_agg = f * cs_ref[...]
    iou = (jnp.dot(hs, wiou_ref[...], preferred_element_type=jnp.float32)
           + biou_ref[...])
    i = jax.nn.sigmoid(iou[:, 0:H_SIZE])
    o = jax.nn.sigmoid(iou[:, H_SIZE:2 * H_SIZE])
    u = jnp.tanh(iou[:, 2 * H_SIZE:3 * H_SIZE])
    c_new = i * u + c_agg
    cn_ref[...] = c_new
    hn_ref[...] = o * jnp.tanh(c_new)


_GATE_BLOCK = 2000


def _gates(h_sum, c_sum, wf_t, bf, wiou_t, biou):
    grid = (N_NODES // _GATE_BLOCK,)
    row_spec = pl.BlockSpec((_GATE_BLOCK, H_SIZE), lambda i: (i, 0))
    iou_w_spec = pl.BlockSpec((H_SIZE, 3 * H_SIZE), lambda i: (0, 0))
    f_w_spec = pl.BlockSpec((H_SIZE, H_SIZE), lambda i: (0, 0))
    return pl.pallas_call(
        _gates_body,
        grid=grid,
        in_specs=[
            row_spec,
            row_spec,
            f_w_spec,
            pl.BlockSpec((1, H_SIZE), lambda i: (0, 0)),
            iou_w_spec,
            pl.BlockSpec((1, 3 * H_SIZE), lambda i: (0, 0)),
        ],
        out_specs=[row_spec, row_spec],
        out_shape=(
            jax.ShapeDtypeStruct((N_NODES, H_SIZE), jnp.float32),
            jax.ShapeDtypeStruct((N_NODES, H_SIZE), jnp.float32),
        ),
    )(h_sum, c_sum, wf_t, bf, wiou_t, biou)


def kernel(h, c, edge_index, U_iou_W, U_f_W, U_f_b, b_iou):
    src = edge_index[0]
    dst = edge_index[1]
    pad = E_PAD - N_EDGES
    src_p = jnp.concatenate([src, jnp.zeros((pad,), jnp.int32)])
    dst_p = jnp.concatenate([dst, jnp.full((pad,), N_NODES, jnp.int32)])
    zeros = jnp.zeros((ZERO_ROWS, H_SIZE), jnp.float32)
    h_sum, c_sum = _segment_sums(h, c, src_p, dst_p, zeros)
    h_new, c_new = _gates(
        h_sum, c_sum,
        U_f_W.T, U_f_b.reshape(1, H_SIZE),
        U_iou_W.T, b_iou.reshape(1, 3 * H_SIZE))
    return (h_new, c_new)


# speedup vs baseline: 1.5522x; 1.5522x over previous
import functools

import jax
import jax.numpy as jnp
from jax import lax
from jax.experimental import pallas as pl
from jax.experimental.pallas import tpu as pltpu
from jax.experimental.pallas import tpu_sc as plsc

N_NODES = 10000
N_EDGES = 320000
H_SIZE = 128

NUM_CORES = 2
NUM_SUBCORES = 16
CHUNK = 128
CHUNKS_PER_SUBCORE = 157
EDGES_PER_SUBCORE = CHUNK * CHUNKS_PER_SUBCORE
E_PAD = EDGES_PER_SUBCORE * NUM_SUBCORES
ACC_ROWS = 10240
ZERO_ROWS = ACC_ROWS // NUM_SUBCORES
OUT_ROWS = 624


def _make_segment_sums():
    mesh = plsc.VectorSubcoreMesh(core_axis_name="c", subcore_axis_name="s")

    @functools.partial(
        pl.kernel,
        mesh=mesh,
        out_type=(
            jax.ShapeDtypeStruct((N_NODES, H_SIZE), jnp.float32),
            jax.ShapeDtypeStruct((N_NODES, H_SIZE), jnp.float32),
        ),
        scratch_types=[
            pltpu.VMEM((CHUNK,), jnp.int32),
            pltpu.VMEM((CHUNK,), jnp.int32),
            pltpu.VMEM((CHUNK, H_SIZE), jnp.float32),
            pltpu.VMEM_SHARED((ACC_ROWS, H_SIZE), jnp.float32),
            pltpu.SemaphoreType.DMA,
        ],
    )
    def seg_sum(h_hbm, c_hbm, src_hbm, dst_hbm, zeros_hbm,
                hsum_hbm, csum_hbm, src_v, dst_v, rows_v, acc, sem):
        cid = lax.axis_index("c")
        sid = lax.axis_index("s")

        pltpu.sync_copy(zeros_hbm, acc.at[pl.ds(sid * ZERO_ROWS, ZERO_ROWS)])
        plsc.subcore_barrier()

        def run_edges(table_hbm):
            @pl.loop(0, CHUNKS_PER_SUBCORE)
            def _(i):
                base = sid * EDGES_PER_SUBCORE + i * CHUNK
                pltpu.sync_copy(src_hbm.at[pl.ds(base, CHUNK)], src_v)
                pltpu.sync_copy(dst_hbm.at[pl.ds(base, CHUNK)], dst_v)
                pltpu.async_copy(table_hbm.at[src_v], rows_v, sem).wait()
                pltpu.sync_copy(rows_v, acc.at[dst_v], add=True)

        @pl.when(cid == 0)
        def _():
            run_edges(h_hbm)

        @pl.when(cid == 1)
        def _():
            run_edges(c_hbm)

        plsc.subcore_barrier()

        def writeout(dst_hbm_ref):
            @pl.when(sid < NUM_SUBCORES - 1)
            def _():
                slc = pl.ds(sid * OUT_ROWS, OUT_ROWS)
                pltpu.sync_copy(acc.at[slc], dst_hbm_ref.at[slc])

            @pl.when(sid == NUM_SUBCORES - 1)
            def _():
                slc = pl.ds((NUM_SUBCORES - 1) * OUT_ROWS,
                            N_NODES - (NUM_SUBCORES - 1) * OUT_ROWS)
                pltpu.sync_copy(acc.at[slc], dst_hbm_ref.at[slc])

        @pl.when(cid == 0)
        def _():
            writeout(hsum_hbm)

        @pl.when(cid == 1)
        def _():
            writeout(csum_hbm)

    return seg_sum


_segment_sums = _make_segment_sums()


def _gates_body(hs_ref, cs_ref, wf_ref, bf_ref, wiou_ref, biou_ref,
                hn_ref, cn_ref):
    hs = hs_ref[...]
    f = jax.nn.sigmoid(
        jnp.dot(hs, wf_ref[...], preferred_element_type=jnp.float32)
        + bf_ref[...])
    c_agg = f * cs_ref[...]
    iou = (jnp.dot(hs, wiou_ref[...], preferred_element_type=jnp.float32)
           + biou_ref[...])
    i = jax.nn.sigmoid(iou[:, 0:H_SIZE])
    o = jax.nn.sigmoid(iou[:, H_SIZE:2 * H_SIZE])
    u = jnp.tanh(iou[:, 2 * H_SIZE:3 * H_SIZE])
    c_new = i * u + c_agg
    cn_ref[...] = c_new
    hn_ref[...] = o * jnp.tanh(c_new)


_GATE_BLOCK = 2000


def _gates(h_sum, c_sum, wf_t, bf, wiou_t, biou):
    grid = (N_NODES // _GATE_BLOCK,)
    row_spec = pl.BlockSpec((_GATE_BLOCK, H_SIZE), lambda i: (i, 0))
    iou_w_spec = pl.BlockSpec((H_SIZE, 3 * H_SIZE), lambda i: (0, 0))
    f_w_spec = pl.BlockSpec((H_SIZE, H_SIZE), lambda i: (0, 0))
    return pl.pallas_call(
        _gates_body,
        grid=grid,
        in_specs=[
            row_spec,
            row_spec,
            f_w_spec,
            pl.BlockSpec((1, H_SIZE), lambda i: (0, 0)),
            iou_w_spec,
            pl.BlockSpec((1, 3 * H_SIZE), lambda i: (0, 0)),
        ],
        out_specs=[row_spec, row_spec],
        out_shape=(
            jax.ShapeDtypeStruct((N_NODES, H_SIZE), jnp.float32),
            jax.ShapeDtypeStruct((N_NODES, H_SIZE), jnp.float32),
        ),
    )(h_sum, c_sum, wf_t, bf, wiou_t, biou)


def kernel(h, c, edge_index, U_iou_W, U_f_W, U_f_b, b_iou):
    src = edge_index[0]
    dst = edge_index[1]
    pad = E_PAD - N_EDGES
    src_p = jnp.concatenate([src, jnp.zeros((pad,), jnp.int32)])
    dst_p = jnp.concatenate([dst, jnp.full((pad,), N_NODES, jnp.int32)])
    zeros = jnp.zeros((ZERO_ROWS, H_SIZE), jnp.float32)
    h_sum, c_sum = _segment_sums(h, c, src_p, dst_p, zeros)
    h_new, c_new = _gates(
        h_sum, c_sum,
        U_f_W.T, U_f_b.reshape(1, H_SIZE),
        U_iou_W.T, b_iou.reshape(1, 3 * H_SIZE))
    return (h_new, c_new)


# speedup vs baseline: 1.9876x; 1.2805x over previous
import functools

import jax
import jax.numpy as jnp
from jax import lax
from jax.experimental import pallas as pl
from jax.experimental.pallas import tpu as pltpu
from jax.experimental.pallas import tpu_sc as plsc

N_NODES = 10000
N_EDGES = 320000
H_SIZE = 128

NUM_CORES = 2
NUM_SUBCORES = 16
CHUNK = 128
CHUNKS_PER_SUBCORE = 157
EDGES_PER_SUBCORE = CHUNK * CHUNKS_PER_SUBCORE
E_PAD = EDGES_PER_SUBCORE * NUM_SUBCORES
ACC_ROWS = 10240
ZERO_ROWS = ACC_ROWS // NUM_SUBCORES
OUT_ROWS = 624


def _make_segment_sums():
    mesh = plsc.VectorSubcoreMesh(core_axis_name="c", subcore_axis_name="s")

    @functools.partial(
        pl.kernel,
        mesh=mesh,
        out_type=(
            jax.ShapeDtypeStruct((N_NODES, H_SIZE), jnp.float32),
            jax.ShapeDtypeStruct((N_NODES, H_SIZE), jnp.float32),
        ),
        scratch_types=[
            pltpu.VMEM((EDGES_PER_SUBCORE,), jnp.int32),
            pltpu.VMEM((CHUNK,), jnp.int32),
            pltpu.VMEM((CHUNK,), jnp.int32),
            pltpu.VMEM((CHUNK, H_SIZE), jnp.float32),
            pltpu.VMEM_SHARED((ACC_ROWS, H_SIZE), jnp.float32),
        ],
    )
    def seg_sum(h_hbm, c_hbm, packed_hbm, zeros_hbm,
                hsum_hbm, csum_hbm, idxp, sbuf, dbuf, rows, acc):
        cid = lax.axis_index("c")
        sid = lax.axis_index("s")

        pltpu.sync_copy(zeros_hbm, acc.at[pl.ds(sid * ZERO_ROWS, ZERO_ROWS)])
        pltpu.sync_copy(packed_hbm.at[sid], idxp)
        plsc.subcore_barrier()

        def run_edges(table_hbm):
            @pl.loop(0, CHUNKS_PER_SUBCORE)
            def _(i):
                for cseg in range(CHUNK // 16):
                    v = idxp[pl.ds(i * CHUNK + cseg * 16, 16)]
                    sbuf[pl.ds(cseg * 16, 16)] = v & 16383
                    dbuf[pl.ds(cseg * 16, 16)] = lax.shift_right_logical(
                        v, 14)
                pltpu.sync_copy(table_hbm.at[sbuf], rows)
                pltpu.sync_copy(rows, acc.at[dbuf], add=True)

        @pl.when(cid == 0)
        def _():
            run_edges(h_hbm)

        @pl.when(cid == 1)
        def _():
            run_edges(c_hbm)

        plsc.subcore_barrier()

        def writeout(dst_hbm_ref):
            @pl.when(sid < NUM_SUBCORES - 1)
            def _():
                slc = pl.ds(sid * OUT_ROWS, OUT_ROWS)
                pltpu.sync_copy(acc.at[slc], dst_hbm_ref.at[slc])

            @pl.when(sid == NUM_SUBCORES - 1)
            def _():
                slc = pl.ds((NUM_SUBCORES - 1) * OUT_ROWS,
                            N_NODES - (NUM_SUBCORES - 1) * OUT_ROWS)
                pltpu.sync_copy(acc.at[slc], dst_hbm_ref.at[slc])

        @pl.when(cid == 0)
        def _():
            writeout(hsum_hbm)

        @pl.when(cid == 1)
        def _():
            writeout(csum_hbm)

    return seg_sum


_segment_sums = _make_segment_sums()


def _gates_body(hs_ref, cs_ref, wf_ref, bf_ref, wiou_ref, biou_ref,
                hn_ref, cn_ref):
    hs = hs_ref[...]
    f = jax.nn.sigmoid(
        jnp.dot(hs, wf_ref[...], preferred_element_type=jnp.float32)
        + bf_ref[...])
    c_agg = f * cs_ref[...]
    iou = (jnp.dot(hs, wiou_ref[...], preferred_element_type=jnp.float32)
           + biou_ref[...])
    i = jax.nn.sigmoid(iou[:, 0:H_SIZE])
    o = jax.nn.sigmoid(iou[:, H_SIZE:2 * H_SIZE])
    u = jnp.tanh(iou[:, 2 * H_SIZE:3 * H_SIZE])
    c_new = i * u + c_agg
    cn_ref[...] = c_new
    hn_ref[...] = o * jnp.tanh(c_new)


_GATE_BLOCK = 2000


def _gates(h_sum, c_sum, wf_t, bf, wiou_t, biou):
    grid = (N_NODES // _GATE_BLOCK,)
    row_spec = pl.BlockSpec((_GATE_BLOCK, H_SIZE), lambda i: (i, 0))
    iou_w_spec = pl.BlockSpec((H_SIZE, 3 * H_SIZE), lambda i: (0, 0))
    f_w_spec = pl.BlockSpec((H_SIZE, H_SIZE), lambda i: (0, 0))
    return pl.pallas_call(
        _gates_body,
        grid=grid,
        in_specs=[
            row_spec,
            row_spec,
            f_w_spec,
            pl.BlockSpec((1, H_SIZE), lambda i: (0, 0)),
            iou_w_spec,
            pl.BlockSpec((1, 3 * H_SIZE), lambda i: (0, 0)),
        ],
        out_specs=[row_spec, row_spec],
        out_shape=(
            jax.ShapeDtypeStruct((N_NODES, H_SIZE), jnp.float32),
            jax.ShapeDtypeStruct((N_NODES, H_SIZE), jnp.float32),
        ),
    )(h_sum, c_sum, wf_t, bf, wiou_t, biou)


def kernel(h, c, edge_index, U_iou_W, U_f_W, U_f_b, b_iou):
    src = edge_index[0]
    dst = edge_index[1]
    pad = E_PAD - N_EDGES
    src_p = jnp.concatenate([src, jnp.zeros((pad,), jnp.int32)])
    dst_p = jnp.concatenate([dst, jnp.full((pad,), N_NODES, jnp.int32)])
    packed = jnp.bitwise_or(src_p, jnp.left_shift(dst_p, 14))
    packed = packed.reshape(NUM_SUBCORES, EDGES_PER_SUBCORE)
    zeros = jnp.zeros((ZERO_ROWS, H_SIZE), jnp.float32)
    h_sum, c_sum = _segment_sums(h, c, packed, zeros)
    h_new, c_new = _gates(
        h_sum, c_sum,
        U_f_W.T, U_f_b.reshape(1, H_SIZE),
        U_iou_W.T, b_iou.reshape(1, 3 * H_SIZE))
    return (h_new, c_new)


# speedup vs baseline: 2.4075x; 1.2112x over previous
import functools

import jax
import jax.numpy as jnp
from jax import lax
from jax.experimental import pallas as pl
from jax.experimental.pallas import tpu as pltpu
from jax.experimental.pallas import tpu_sc as plsc

N_NODES = 10000
N_EDGES = 320000
H_SIZE = 128

NUM_CORES = 2
NUM_SUBCORES = 16
CHUNK = 112
CHUNKS_PER_SUBCORE = 179
EDGES_PER_SUBCORE = CHUNK * CHUNKS_PER_SUBCORE
E_PAD = EDGES_PER_SUBCORE * NUM_SUBCORES
ACC_ROWS = 10112
ZERO_ROWS = ACC_ROWS // NUM_SUBCORES
OUT_ROWS = 624


def _make_segment_sums():
    mesh = plsc.VectorSubcoreMesh(core_axis_name="c", subcore_axis_name="s")

    @functools.partial(
        pl.kernel,
        mesh=mesh,
        out_type=(
            jax.ShapeDtypeStruct((N_NODES, H_SIZE), jnp.float32),
            jax.ShapeDtypeStruct((N_NODES, H_SIZE), jnp.float32),
        ),
        scratch_types=[
            pltpu.VMEM((EDGES_PER_SUBCORE,), jnp.int32),
            pltpu.VMEM((CHUNK,), jnp.int32),
            pltpu.VMEM((CHUNK,), jnp.int32),
            pltpu.VMEM((CHUNK,), jnp.int32),
            pltpu.VMEM((CHUNK,), jnp.int32),
            pltpu.VMEM((CHUNK, H_SIZE), jnp.float32),
            pltpu.VMEM((CHUNK, H_SIZE), jnp.float32),
            pltpu.VMEM_SHARED((ACC_ROWS, H_SIZE), jnp.float32),
            pltpu.SemaphoreType.DMA,
            pltpu.SemaphoreType.DMA,
        ],
    )
    def seg_sum(h_hbm, c_hbm, packed_hbm, zeros_hbm,
                hsum_hbm, csum_hbm, idxp, s0, s1, d0, d1, r0, r1, acc,
                semA, semB):
        cid = lax.axis_index("c")
        sid = lax.axis_index("s")

        pltpu.sync_copy(zeros_hbm, acc.at[pl.ds(sid * ZERO_ROWS, ZERO_ROWS)])
        pltpu.sync_copy(packed_hbm.at[sid], idxp)
        plsc.subcore_barrier()

        def unpack(i, sb, db):
            for cseg in range(CHUNK // 16):
                v = idxp[pl.ds(i * CHUNK + cseg * 16, 16)]
                sb[pl.ds(cseg * 16, 16)] = v & 16383
                db[pl.ds(cseg * 16, 16)] = lax.shift_right_logical(v, 14)

        def run_edges(table_hbm):
            @pl.loop(0, CHUNKS_PER_SUBCORE // 2)
            def _(j):
                i0 = j * 2
                unpack(i0, s0, d0)
                unpack(i0 + 1, s1, d1)
                hA = pltpu.async_copy(table_hbm.at[s0], r0, semA)
                hB = pltpu.async_copy(table_hbm.at[s1], r1, semB)
                hA.wait()
                pltpu.sync_copy(r0, acc.at[d0], add=True)
                hB.wait()
                pltpu.sync_copy(r1, acc.at[d1], add=True)

            i_last = CHUNKS_PER_SUBCORE - 1
            unpack(i_last, s0, d0)
            pltpu.sync_copy(table_hbm.at[s0], r0)
            pltpu.sync_copy(r0, acc.at[d0], add=True)

        @pl.when(cid == 0)
        def _():
            run_edges(h_hbm)

        @pl.when(cid == 1)
        def _():
            run_edges(c_hbm)

        plsc.subcore_barrier()

        def writeout(dst_hbm_ref):
            @pl.when(sid < NUM_SUBCORES - 1)
            def _():
                slc = pl.ds(sid * OUT_ROWS, OUT_ROWS)
                pltpu.sync_copy(acc.at[slc], dst_hbm_ref.at[slc])

            @pl.when(sid == NUM_SUBCORES - 1)
            def _():
                slc = pl.ds((NUM_SUBCORES - 1) * OUT_ROWS,
                            N_NODES - (NUM_SUBCORES - 1) * OUT_ROWS)
                pltpu.sync_copy(acc.at[slc], dst_hbm_ref.at[slc])

        @pl.when(cid == 0)
        def _():
            writeout(hsum_hbm)

        @pl.when(cid == 1)
        def _():
            writeout(csum_hbm)

    return seg_sum


_segment_sums = _make_segment_sums()


def _gates_body(hs_ref, cs_ref, wf_ref, bf_ref, wiou_ref, biou_ref,
                hn_ref, cn_ref):
    hs = hs_ref[...]
    f = jax.nn.sigmoid(
        jnp.dot(hs, wf_ref[...], preferred_element_type=jnp.float32)
        + bf_ref[...])
    c_agg = f * cs_ref[...]
    iou = (jnp.dot(hs, wiou_ref[...], preferred_element_type=jnp.float32)
           + biou_ref[...])
    i = jax.nn.sigmoid(iou[:, 0:H_SIZE])
    o = jax.nn.sigmoid(iou[:, H_SIZE:2 * H_SIZE])
    u = jnp.tanh(iou[:, 2 * H_SIZE:3 * H_SIZE])
    c_new = i * u + c_agg
    cn_ref[...] = c_new
    hn_ref[...] = o * jnp.tanh(c_new)


_GATE_BLOCK = 2000


def _gates(h_sum, c_sum, wf_t, bf, wiou_t, biou):
    grid = (N_NODES // _GATE_BLOCK,)
    row_spec = pl.BlockSpec((_GATE_BLOCK, H_SIZE), lambda i: (i, 0))
    iou_w_spec = pl.BlockSpec((H_SIZE, 3 * H_SIZE), lambda i: (0, 0))
    f_w_spec = pl.BlockSpec((H_SIZE, H_SIZE), lambda i: (0, 0))
    return pl.pallas_call(
        _gates_body,
        grid=grid,
        in_specs=[
            row_spec,
            row_spec,
            f_w_spec,
            pl.BlockSpec((1, H_SIZE), lambda i: (0, 0)),
            iou_w_spec,
            pl.BlockSpec((1, 3 * H_SIZE), lambda i: (0, 0)),
        ],
        out_specs=[row_spec, row_spec],
        out_shape=(
            jax.ShapeDtypeStruct((N_NODES, H_SIZE), jnp.float32),
            jax.ShapeDtypeStruct((N_NODES, H_SIZE), jnp.float32),
        ),
    )(h_sum, c_sum, wf_t, bf, wiou_t, biou)


def kernel(h, c, edge_index, U_iou_W, U_f_W, U_f_b, b_iou):
    src = edge_index[0]
    dst = edge_index[1]
    pad = E_PAD - N_EDGES
    src_p = jnp.concatenate([src, jnp.zeros((pad,), jnp.int32)])
    dst_p = jnp.concatenate([dst, jnp.full((pad,), N_NODES, jnp.int32)])
    packed = jnp.bitwise_or(src_p, jnp.left_shift(dst_p, 14))
    packed = packed.reshape(NUM_SUBCORES, EDGES_PER_SUBCORE)
    zeros = jnp.zeros((ZERO_ROWS, H_SIZE), jnp.float32)
    h_sum, c_sum = _segment_sums(h, c, packed, zeros)
    h_new, c_new = _gates(
        h_sum, c_sum,
        U_f_W.T, U_f_b.reshape(1, H_SIZE),
        U_iou_W.T, b_iou.reshape(1, 3 * H_SIZE))
    return (h_new, c_new)


# speedup vs baseline: 2.4341x; 1.0111x over previous
import functools

import jax
import jax.numpy as jnp
from jax import lax
from jax.experimental import pallas as pl
from jax.experimental.pallas import tpu as pltpu
from jax.experimental.pallas import tpu_sc as plsc

N_NODES = 10000
N_EDGES = 320000
H_SIZE = 128

NUM_CORES = 2
NUM_SUBCORES = 16
CHUNK = 112
CHUNKS_PER_SUBCORE = 179
EDGES_PER_SUBCORE = CHUNK * CHUNKS_PER_SUBCORE
E_PAD = EDGES_PER_SUBCORE * NUM_SUBCORES
ACC_ROWS = 10112
ZERO_ROWS = ACC_ROWS // NUM_SUBCORES
OUT_ROWS = 624


def _make_segment_sums():
    mesh = plsc.VectorSubcoreMesh(core_axis_name="c", subcore_axis_name="s")

    @functools.partial(
        pl.kernel,
        mesh=mesh,
        out_type=(
            jax.ShapeDtypeStruct((N_NODES, H_SIZE), jnp.float32),
            jax.ShapeDtypeStruct((N_NODES, H_SIZE), jnp.float32),
        ),
        scratch_types=[
            pltpu.VMEM((EDGES_PER_SUBCORE,), jnp.int32),
            pltpu.VMEM((CHUNK,), jnp.int32),
            pltpu.VMEM((CHUNK,), jnp.int32),
            pltpu.VMEM((CHUNK,), jnp.int32),
            pltpu.VMEM((CHUNK,), jnp.int32),
            pltpu.VMEM((CHUNK, H_SIZE), jnp.float32),
            pltpu.VMEM((CHUNK, H_SIZE), jnp.float32),
            pltpu.VMEM_SHARED((ACC_ROWS, H_SIZE), jnp.float32),
            pltpu.SemaphoreType.DMA,
            pltpu.SemaphoreType.DMA,
            pltpu.SemaphoreType.DMA,
            pltpu.SemaphoreType.DMA,
        ],
    )
    def seg_sum(h_hbm, c_hbm, packed_hbm, zeros_hbm,
                hsum_hbm, csum_hbm, idxp, s0, s1, d0, d1, r0, r1, acc,
                semA, semB, semA2, semB2):
        cid = lax.axis_index("c")
        sid = lax.axis_index("s")

        pltpu.sync_copy(zeros_hbm, acc.at[pl.ds(sid * ZERO_ROWS, ZERO_ROWS)])
        pltpu.sync_copy(packed_hbm.at[sid], idxp)
        plsc.subcore_barrier()

        def unpack(i, sb, db):
            for cseg in range(CHUNK // 16):
                v = idxp[pl.ds(i * CHUNK + cseg * 16, 16)]
                sb[pl.ds(cseg * 16, 16)] = v & 16383
                db[pl.ds(cseg * 16, 16)] = lax.shift_right_logical(v, 14)

        def run_edges(table_hbm):
            def drain_scatters():
                pltpu.make_async_copy(r0, acc.at[d0], semA2).wait()
                pltpu.make_async_copy(r1, acc.at[d1], semB2).wait()

            @pl.loop(0, CHUNKS_PER_SUBCORE // 2)
            def _(j):
                @pl.when(j > 0)
                def _():
                    drain_scatters()
                i0 = j * 2
                unpack(i0, s0, d0)
                unpack(i0 + 1, s1, d1)
                hA = pltpu.async_copy(table_hbm.at[s0], r0, semA)
                hB = pltpu.async_copy(table_hbm.at[s1], r1, semB)
                hA.wait()
                pltpu.async_copy(r0, acc.at[d0], semA2, add=True)
                hB.wait()
                pltpu.async_copy(r1, acc.at[d1], semB2, add=True)

            drain_scatters()
            i_last = CHUNKS_PER_SUBCORE - 1
            unpack(i_last, s0, d0)
            pltpu.sync_copy(table_hbm.at[s0], r0)
            pltpu.sync_copy(r0, acc.at[d0], add=True)

        @pl.when(cid == 0)
        def _():
            run_edges(h_hbm)

        @pl.when(cid == 1)
        def _():
            run_edges(c_hbm)

        plsc.subcore_barrier()

        def writeout(dst_hbm_ref):
            @pl.when(sid < NUM_SUBCORES - 1)
            def _():
                slc = pl.ds(sid * OUT_ROWS, OUT_ROWS)
                pltpu.sync_copy(acc.at[slc], dst_hbm_ref.at[slc])

            @pl.when(sid == NUM_SUBCORES - 1)
            def _():
                slc = pl.ds((NUM_SUBCORES - 1) * OUT_ROWS,
                            N_NODES - (NUM_SUBCORES - 1) * OUT_ROWS)
                pltpu.sync_copy(acc.at[slc], dst_hbm_ref.at[slc])

        @pl.when(cid == 0)
        def _():
            writeout(hsum_hbm)

        @pl.when(cid == 1)
        def _():
            writeout(csum_hbm)

    return seg_sum


_segment_sums = _make_segment_sums()


def _gates_body(hs_ref, cs_ref, wf_ref, bf_ref, wiou_ref, biou_ref,
                hn_ref, cn_ref):
    hs = hs_ref[...]
    f = jax.nn.sigmoid(
        jnp.dot(hs, wf_ref[...], preferred_element_type=jnp.float32)
        + bf_ref[...])
    c_agg = f * cs_ref[...]
    iou = (jnp.dot(hs, wiou_ref[...], preferred_element_type=jnp.float32)
           + biou_ref[...])
    i = jax.nn.sigmoid(iou[:, 0:H_SIZE])
    o = jax.nn.sigmoid(iou[:, H_SIZE:2 * H_SIZE])
    u = jnp.tanh(iou[:, 2 * H_SIZE:3 * H_SIZE])
    c_new = i * u + c_agg
    cn_ref[...] = c_new
    hn_ref[...] = o * jnp.tanh(c_new)


_GATE_BLOCK = 2000


def _gates(h_sum, c_sum, wf_t, bf, wiou_t, biou):
    grid = (N_NODES // _GATE_BLOCK,)
    row_spec = pl.BlockSpec((_GATE_BLOCK, H_SIZE), lambda i: (i, 0))
    iou_w_spec = pl.BlockSpec((H_SIZE, 3 * H_SIZE), lambda i: (0, 0))
    f_w_spec = pl.BlockSpec((H_SIZE, H_SIZE), lambda i: (0, 0))
    return pl.pallas_call(
        _gates_body,
        grid=grid,
        in_specs=[
            row_spec,
            row_spec,
            f_w_spec,
            pl.BlockSpec((1, H_SIZE), lambda i: (0, 0)),
            iou_w_spec,
            pl.BlockSpec((1, 3 * H_SIZE), lambda i: (0, 0)),
        ],
        out_specs=[row_spec, row_spec],
        out_shape=(
            jax.ShapeDtypeStruct((N_NODES, H_SIZE), jnp.float32),
            jax.ShapeDtypeStruct((N_NODES, H_SIZE), jnp.float32),
        ),
    )(h_sum, c_sum, wf_t, bf, wiou_t, biou)


def kernel(h, c, edge_index, U_iou_W, U_f_W, U_f_b, b_iou):
    src = edge_index[0]
    dst = edge_index[1]
    pad = E_PAD - N_EDGES
    src_p = jnp.concatenate([src, jnp.zeros((pad,), jnp.int32)])
    dst_p = jnp.concatenate([dst, jnp.full((pad,), N_NODES, jnp.int32)])
    packed = jnp.bitwise_or(src_p, jnp.left_shift(dst_p, 14))
    packed = packed.reshape(NUM_SUBCORES, EDGES_PER_SUBCORE)
    zeros = jnp.zeros((ZERO_ROWS, H_SIZE), jnp.float32)
    h_sum, c_sum = _segment_sums(h, c, packed, zeros)
    h_new, c_new = _gates(
        h_sum, c_sum,
        U_f_W.T, U_f_b.reshape(1, H_SIZE),
        U_iou_W.T, b_iou.reshape(1, 3 * H_SIZE))
    return (h_new, c_new)
